# Initial kernel scaffold; baseline (speedup 1.0000x reference)
import functools

import jax
import jax.numpy as jnp
from jax import lax
from jax.experimental import pallas as pl
from jax.experimental.pallas import tpu as pltpu
from jax.experimental.pallas import tpu_sc as plsc

N = 10000
E = 320000
NFEAT = 128
NHID = 8
NHEADS = 8
NCLASS = 16
ALPHA = 0.2

NC = 2
NS = 16
B = 128
NB = E // B
RPS = N // NS


_BLK = 2000
_GRID = N // _BLK


def _tc1_body(x_ref, wp_ref, al_ref, ar_ref, hp_ref, sl_ref):
  xb = x_ref[...]
  hp80 = jnp.dot(xb, wp_ref[...], preferred_element_type=jnp.float32)
  col = lax.broadcasted_iota(jnp.int32, hp80.shape, 1)
  hp80 = hp80 + jnp.where(col % 10 == 8, 1.0, 0.0).astype(jnp.float32)
  sl = jnp.dot(hp80, al_ref[...], preferred_element_type=jnp.float32)
  sr = jnp.dot(hp80, ar_ref[...], preferred_element_type=jnp.float32)
  pad = jnp.zeros((hp80.shape[0], 8), jnp.float32)
  hp_ref[...] = jnp.concatenate([hp80, sr, pad], axis=1)
  sl_ref[...] = sl


def _tc1(x, wp, al, ar):
  return pl.pallas_call(
      _tc1_body,
      grid=(_GRID,),
      in_specs=[
          pl.BlockSpec((_BLK, NFEAT), lambda i: (i, 0)),
          pl.BlockSpec((NFEAT, 80), lambda i: (0, 0)),
          pl.BlockSpec((80, NHEADS), lambda i: (0, 0)),
          pl.BlockSpec((80, NHEADS), lambda i: (0, 0)),
      ],
      out_specs=[
          pl.BlockSpec((_BLK, 96), lambda i: (i, 0)),
          pl.BlockSpec((_BLK, NHEADS), lambda i: (i, 0)),
      ],
      out_shape=[
          jax.ShapeDtypeStruct((N, 96), jnp.float32),
          jax.ShapeDtypeStruct((N, NHEADS), jnp.float32),
      ],
  )(x, wp, al, ar)


def _tc2_body(acc_ref, wo_ref, ao_ref, hp2_ref, sl2_ref):
  A = acc_ref[0] + acc_ref[1]
  cols = []
  for i in range(NHEADS):
    num = A[:, 10 * i:10 * i + 8]
    den = A[:, 10 * i + 8:10 * i + 9] + 1e-16
    v = num / den
    cols.append(jnp.where(v > 0, v, jnp.exp(v) - 1.0))
  xc = jnp.concatenate(cols, axis=1)
  h2 = jnp.dot(xc, wo_ref[...], preferred_element_type=jnp.float32)
  ao = ao_ref[...]
  sl2 = jnp.sum(h2 * ao[:, :NCLASS], axis=1, keepdims=True)
  sr2 = jnp.sum(h2 * ao[:, NCLASS:], axis=1, keepdims=True)
  one = jnp.ones((h2.shape[0], 1), jnp.float32)
  hp2_ref[...] = jnp.concatenate(
      [h2, one, sr2, jnp.zeros((h2.shape[0], 14), jnp.float32)], axis=1)
  sl2_ref[...] = jnp.concatenate(
      [sl2, jnp.zeros((h2.shape[0], 7), jnp.float32)], axis=1)


def _tc2(acc, wo, ao):
  return pl.pallas_call(
      _tc2_body,
      grid=(_GRID,),
      in_specs=[
          pl.BlockSpec((2, _BLK, 80), lambda i: (0, i, 0)),
          pl.BlockSpec((NHID * NHEADS, NCLASS), lambda i: (0, 0)),
          pl.BlockSpec((1, 2 * NCLASS), lambda i: (0, 0)),
      ],
      out_specs=[
          pl.BlockSpec((_BLK, 32), lambda i: (i, 0)),
          pl.BlockSpec((_BLK, 8), lambda i: (i, 0)),
      ],
      out_shape=[
          jax.ShapeDtypeStruct((N, 32), jnp.float32),
          jax.ShapeDtypeStruct((N, 8), jnp.float32),
      ],
  )(acc, wo, ao)


def _tc3_body(acc_ref, out_ref):
  A = acc_ref[0] + acc_ref[1]
  out_ref[...] = A[:, :NCLASS] / (A[:, NCLASS:NCLASS + 1] + 1e-16)


def _tc3(acc):
  return pl.pallas_call(
      _tc3_body,
      grid=(_GRID,),
      in_specs=[pl.BlockSpec((2, _BLK, 24), lambda i: (0, i, 0))],
      out_specs=pl.BlockSpec((_BLK, NCLASS), lambda i: (i, 0)),
      out_shape=jax.ShapeDtypeStruct((N, NCLASS), jnp.float32),
  )(acc)


def _make_sc_edge(cw, sw, heads):
  def body(row_h, col_h, hp_h, sl_h, out_h, rv, cv, slv, hv, cb, acc, sem1,
           sem2):
    c = lax.axis_index("c")
    s = lax.axis_index("s")
    w = c * NS + s
    lanes = lax.iota(jnp.int32, 16)
    zero16 = jnp.zeros((16,), jnp.float32)

    def zb(j, carry):
      flat = j * 16 + lanes
      plsc.store_scatter(cb, [flat // sw, flat % sw], zero16)
      return carry

    lax.fori_loop(0, B * sw // 16, zb, 0)

    r0 = s * RPS
    for off in range(0, RPS, B):
      sz = min(B, RPS - off)
      pltpu.sync_copy(cb.at[pl.ds(0, sz)], acc.at[pl.ds(r0 + off, sz)])
    plsc.subcore_barrier()

    def batch_body(t, carry):
      b = w + t * (NC * NS)

      @pl.when(b < NB)
      def _():
        base = b * B
        pltpu.sync_copy(row_h.at[pl.ds(base, B)], rv)
        pltpu.sync_copy(col_h.at[pl.ds(base, B)], cv)
        d1 = pltpu.async_copy(sl_h.at[rv], slv, sem1)
        d2 = pltpu.async_copy(hp_h.at[cv], hv, sem2)
        d1.wait()
        d2.wait()
        for g in range(B // 16):
          e = g * 16 + lanes
          for (slc, src, pairs, dcol) in heads:
            s1 = plsc.load_gather(slv, [e, jnp.full((16,), slc, jnp.int32)])
            s2 = plsc.load_gather(hv, [e, jnp.full((16,), src, jnp.int32)])
            v = s1 + s2
            lr = jnp.where(v > 0, v, ALPHA * v)
            wgt = jnp.exp(-lr)
            for (hcol, ocol) in pairs:
              hc = plsc.load_gather(hv, [e, jnp.full((16,), hcol, jnp.int32)])
              plsc.store_scatter(cb, [e, jnp.full((16,), ocol, jnp.int32)],
                                 wgt * hc)
            plsc.store_scatter(cb, [e, jnp.full((16,), dcol, jnp.int32)], wgt)
        pltpu.sync_copy(cb, acc.at[rv], add=True)

      return carry

    lax.fori_loop(0, (NB + NC * NS - 1) // (NC * NS), batch_body, 0)
    plsc.subcore_barrier()
    pltpu.sync_copy(acc.at[pl.ds(r0, RPS)], out_h.at[c, pl.ds(r0, RPS)])

  mesh = plsc.VectorSubcoreMesh(
      core_axis_name="c", subcore_axis_name="s", num_cores=NC, num_subcores=NS)
  return pl.kernel(
      body,
      out_type=jax.ShapeDtypeStruct((NC, N, sw), jnp.float32),
      mesh=mesh,
      scratch_types=[
          pltpu.VMEM((B,), jnp.int32),
          pltpu.VMEM((B,), jnp.int32),
          pltpu.VMEM((B, 8), jnp.float32),
          pltpu.VMEM((B, cw), jnp.float32),
          pltpu.VMEM((B, sw), jnp.float32),
          pltpu.VMEM_SHARED((N, sw), jnp.float32),
          pltpu.SemaphoreType.DMA,
          pltpu.SemaphoreType.DMA,
      ],
  )


_HEADS1 = [
    (i, 80 + i, [(10 * i + k, 10 * i + k) for k in range(NHID)], 10 * i + 8)
    for i in range(NHEADS)
]
_HEADS2 = [(0, NCLASS + 1, [(k, k) for k in range(NCLASS)], NCLASS)]

_sc_edge1 = _make_sc_edge(96, 80, _HEADS1)
_sc_edge2 = _make_sc_edge(32, 24, _HEADS2)


@jax.jit
def kernel(x, edge_index, W, a, W_out, a_out):
  row = edge_index[0]
  col = edge_index[1]

  wt = jnp.transpose(W, (1, 0, 2))
  wp = jnp.pad(wt, ((0, 0), (0, 0), (0, 2))).reshape(NFEAT, 80)
  I = jnp.arange(NHEADS)
  K = jnp.arange(NHID)
  al = jnp.zeros((NHEADS, 10, NHEADS), jnp.float32)
  al = al.at[I[:, None], K[None, :], I[:, None]].set(a[:, :NHID])
  al = al.reshape(80, NHEADS)
  ar = jnp.zeros((NHEADS, 10, NHEADS), jnp.float32)
  ar = ar.at[I[:, None], K[None, :], I[:, None]].set(a[:, NHID:])
  ar = ar.reshape(80, NHEADS)

  hp, sl = _tc1(x, wp, al, ar)
  acc1 = _sc_edge1(row, col, hp, sl)
  hp2, sl2 = _tc2(acc1, W_out, a_out.reshape(1, 2 * NCLASS))
  acc2 = _sc_edge2(row, col, hp2, sl2)
  return _tc3(acc2)


# speedup vs baseline: 9.6303x; 9.6303x over previous
import jax
import jax.numpy as jnp
from jax import lax
from jax.experimental import pallas as pl
from jax.experimental.pallas import tpu as pltpu
from jax.experimental.pallas import tpu_sc as plsc

N = 10000
E = 320000
NFEAT = 128
NHID = 8
NHEADS = 8
NCLASS = 16
ALPHA = 0.2

NC = 2
NS = 16
B = 128
NB = E // B
RPS = 624
REM = N - NS * RPS


_BLK = 2000
_GRID = N // _BLK


def _tc1_body(x_ref, wp_ref, al_ref, ar_ref, hp_ref):
  xb = x_ref[...]
  hp80 = jnp.dot(xb, wp_ref[...], preferred_element_type=jnp.float32)
  col = lax.broadcasted_iota(jnp.int32, hp80.shape, 1)
  hp80 = hp80 + jnp.where(col % 10 == 8, 1.0, 0.0).astype(jnp.float32)
  sl = jnp.dot(hp80, al_ref[...], preferred_element_type=jnp.float32)
  sr = jnp.dot(hp80, ar_ref[...], preferred_element_type=jnp.float32)
  pad = jnp.zeros((hp80.shape[0], 32), jnp.float32)
  hp_ref[...] = jnp.concatenate([hp80, sr, sl, pad], axis=1)


def _tc1(x, wp, al, ar):
  return pl.pallas_call(
      _tc1_body,
      grid=(_GRID,),
      in_specs=[
          pl.BlockSpec((_BLK, NFEAT), lambda i: (i, 0)),
          pl.BlockSpec((NFEAT, 80), lambda i: (0, 0)),
          pl.BlockSpec((80, NHEADS), lambda i: (0, 0)),
          pl.BlockSpec((80, NHEADS), lambda i: (0, 0)),
      ],
      out_specs=pl.BlockSpec((_BLK, 128), lambda i: (i, 0)),
      out_shape=jax.ShapeDtypeStruct((N, 128), jnp.float32),
  )(x, wp, al, ar)


def _tc2_body(acc_ref, wo_ref, ao_ref, hp2_ref):
  A = acc_ref[0] + acc_ref[1]
  cols = []
  for i in range(NHEADS):
    num = A[:, 10 * i:10 * i + 8]
    den = A[:, 10 * i + 8:10 * i + 9] + 1e-16
    v = num / den
    cols.append(jnp.where(v > 0, v, jnp.exp(v) - 1.0))
  xc = jnp.concatenate(cols, axis=1)
  h2 = jnp.dot(xc, wo_ref[...], preferred_element_type=jnp.float32)
  ao = ao_ref[...]
  sl2 = jnp.sum(h2 * ao[:, :NCLASS], axis=1, keepdims=True)
  sr2 = jnp.sum(h2 * ao[:, NCLASS:], axis=1, keepdims=True)
  one = jnp.ones((h2.shape[0], 1), jnp.float32)
  hp2_ref[...] = jnp.concatenate(
      [h2, one, sr2, sl2, jnp.zeros((h2.shape[0], 109), jnp.float32)], axis=1)


def _tc2(acc, wo, ao):
  return pl.pallas_call(
      _tc2_body,
      grid=(_GRID,),
      in_specs=[
          pl.BlockSpec((2, _BLK, 128), lambda i: (0, i, 0)),
          pl.BlockSpec((NHID * NHEADS, NCLASS), lambda i: (0, 0)),
          pl.BlockSpec((1, 2 * NCLASS), lambda i: (0, 0)),
      ],
      out_specs=pl.BlockSpec((_BLK, 128), lambda i: (i, 0)),
      out_shape=jax.ShapeDtypeStruct((N, 128), jnp.float32),
  )(acc, wo, ao)


def _tc3_body(acc_ref, out_ref):
  A = acc_ref[0] + acc_ref[1]
  out_ref[...] = A[:, :NCLASS] / (A[:, NCLASS:NCLASS + 1] + 1e-16)


def _tc3(acc):
  return pl.pallas_call(
      _tc3_body,
      grid=(_GRID,),
      in_specs=[pl.BlockSpec((2, _BLK, 128), lambda i: (0, i, 0))],
      out_specs=pl.BlockSpec((_BLK, NCLASS), lambda i: (i, 0)),
      out_shape=jax.ShapeDtypeStruct((N, NCLASS), jnp.float32),
  )(acc)


def _make_sc_edge(heads):
  def body(row_h, col_h, hp_h, z_h, out_h, rv, cv, hv, hvr, cb, acc, sem1,
           sem2):
    c = lax.axis_index("c")
    s = lax.axis_index("s")
    w = c * NS + s
    lanes = lax.iota(jnp.int32, 16)

    pltpu.sync_copy(z_h, cb)
    r0 = s * RPS
    for off in range(0, RPS, B):
      sz = min(B, RPS - off)
      pltpu.sync_copy(cb.at[pl.ds(0, sz)], acc.at[pl.ds(r0 + off, sz)])

    @pl.when(s == NS - 1)
    def _():
      pltpu.sync_copy(cb.at[pl.ds(0, REM)], acc.at[pl.ds(NS * RPS, REM)])

    plsc.subcore_barrier()

    def batch_body(t, carry):
      b = w + t * (NC * NS)

      @pl.when(b < NB)
      def _():
        pltpu.sync_copy(row_h.at[pl.ds(b, 1)], rv)
        pltpu.sync_copy(col_h.at[pl.ds(b, 1)], cv)
        d1 = pltpu.async_copy(hp_h.at[cv.at[0]], hv, sem1)
        d2 = pltpu.async_copy(hp_h.at[rv.at[0]], hvr, sem2)
        d1.wait()
        d2.wait()
        for g in range(B // 16):
          e = g * 16 + lanes
          for (slc, src, pairs, dcol) in heads:
            s1 = plsc.load_gather(hvr, [e, jnp.full((16,), slc, jnp.int32)])
            s2 = plsc.load_gather(hv, [e, jnp.full((16,), src, jnp.int32)])
            v = s1 + s2
            lr = jnp.where(v > 0, v, ALPHA * v)
            wgt = jnp.exp(-lr)
            for (hcol, ocol) in pairs:
              hc = plsc.load_gather(hv, [e, jnp.full((16,), hcol, jnp.int32)])
              plsc.store_scatter(cb, [e, jnp.full((16,), ocol, jnp.int32)],
                                 wgt * hc)
            plsc.store_scatter(cb, [e, jnp.full((16,), dcol, jnp.int32)], wgt)
        for g in range(B // 16):
          idxv = rv[0, pl.ds(g * 16, 16)]
          pltpu.async_copy(cb.at[pl.ds(g * 16, 16)], acc.at[idxv], sem1,
                           add=True).wait()

      return carry

    lax.fori_loop(0, (NB + NC * NS - 1) // (NC * NS), batch_body, 0)
    plsc.subcore_barrier()
    pltpu.sync_copy(acc.at[pl.ds(r0, RPS)], out_h.at[c, pl.ds(r0, RPS)])

    @pl.when(s == NS - 1)
    def _():
      pltpu.sync_copy(acc.at[pl.ds(NS * RPS, REM)],
                      out_h.at[c, pl.ds(NS * RPS, REM)])

  mesh = plsc.VectorSubcoreMesh(
      core_axis_name="c", subcore_axis_name="s", num_cores=NC, num_subcores=NS)
  return pl.kernel(
      body,
      out_type=jax.ShapeDtypeStruct((NC, N, 128), jnp.float32),
      mesh=mesh,
      compiler_params=pltpu.CompilerParams(needs_layout_passes=False),
      scratch_types=[
          pltpu.VMEM((1, B), jnp.int32),
          pltpu.VMEM((1, B), jnp.int32),
          pltpu.VMEM((B, 128), jnp.float32),
          pltpu.VMEM((B, 128), jnp.float32),
          pltpu.VMEM((B, 128), jnp.float32),
          pltpu.VMEM_SHARED((N, 128), jnp.float32),
          pltpu.SemaphoreType.DMA,
          pltpu.SemaphoreType.DMA,
      ],
  )


_HEADS1 = [
    (88 + i, 80 + i, [(10 * i + k, 10 * i + k) for k in range(NHID)],
     10 * i + 8)
    for i in range(NHEADS)
]
_HEADS2 = [(NCLASS + 2, NCLASS + 1, [(k, k) for k in range(NCLASS)], NCLASS)]

_sc_edge1 = _make_sc_edge(_HEADS1)
_sc_edge2 = _make_sc_edge(_HEADS2)


@jax.jit
def kernel(x, edge_index, W, a, W_out, a_out):
  row = edge_index[0]
  col = edge_index[1]

  wt = jnp.transpose(W, (1, 0, 2))
  wp = jnp.pad(wt, ((0, 0), (0, 0), (0, 2))).reshape(NFEAT, 80)
  I = jnp.arange(NHEADS)
  K = jnp.arange(NHID)
  al = jnp.zeros((NHEADS, 10, NHEADS), jnp.float32)
  al = al.at[I[:, None], K[None, :], I[:, None]].set(a[:, :NHID])
  al = al.reshape(80, NHEADS)
  ar = jnp.zeros((NHEADS, 10, NHEADS), jnp.float32)
  ar = ar.at[I[:, None], K[None, :], I[:, None]].set(a[:, NHID:])
  ar = ar.reshape(80, NHEADS)

  row2 = row.reshape(NB, B)
  col2 = col.reshape(NB, B)
  z = jnp.zeros((B, 128), jnp.float32)
  hp = _tc1(x, wp, al, ar)
  acc1 = _sc_edge1(row2, col2, hp, z)
  hp2 = _tc2(acc1, W_out, a_out.reshape(1, 2 * NCLASS))
  acc2 = _sc_edge2(row2, col2, hp2, z)
  return _tc3(acc2)


# speedup vs baseline: 9.9800x; 1.0363x over previous
import jax
import jax.numpy as jnp
from jax import lax
from jax.experimental import pallas as pl
from jax.experimental.pallas import tpu as pltpu
from jax.experimental.pallas import tpu_sc as plsc

N = 10000
E = 320000
NFEAT = 128
NHID = 8
NHEADS = 8
NCLASS = 16
ALPHA = 0.2

NC = 2
NS = 16
B = 128
NB = E // B
RPS = 624
REM = N - NS * RPS


_BLK = 2000
_GRID = N // _BLK


def _tc1_body(x_ref, wp_ref, al_ref, ar_ref, hp_ref):
  xb = x_ref[...]
  hp80 = jnp.dot(xb, wp_ref[...], preferred_element_type=jnp.float32)
  col = lax.broadcasted_iota(jnp.int32, hp80.shape, 1)
  hp80 = hp80 + jnp.where(col % 10 == 8, 1.0, 0.0).astype(jnp.float32)
  sl = jnp.dot(hp80, al_ref[...], preferred_element_type=jnp.float32)
  sr = jnp.dot(hp80, ar_ref[...], preferred_element_type=jnp.float32)
  pad = jnp.zeros((hp80.shape[0], 32), jnp.float32)
  hp_ref[...] = jnp.concatenate([hp80, sr, sl, pad], axis=1)


def _tc1(x, wp, al, ar):
  return pl.pallas_call(
      _tc1_body,
      grid=(_GRID,),
      in_specs=[
          pl.BlockSpec((_BLK, NFEAT), lambda i: (i, 0)),
          pl.BlockSpec((NFEAT, 80), lambda i: (0, 0)),
          pl.BlockSpec((80, NHEADS), lambda i: (0, 0)),
          pl.BlockSpec((80, NHEADS), lambda i: (0, 0)),
      ],
      out_specs=pl.BlockSpec((_BLK, 128), lambda i: (i, 0)),
      out_shape=jax.ShapeDtypeStruct((N, 128), jnp.float32),
  )(x, wp, al, ar)


def _tc2_body(acc_ref, wo_ref, ao_ref, hp2_ref):
  A = acc_ref[0] + acc_ref[1]
  cols = []
  for i in range(NHEADS):
    num = A[:, 10 * i:10 * i + 8]
    den = A[:, 10 * i + 8:10 * i + 9] + 1e-16
    v = num / den
    cols.append(jnp.where(v > 0, v, jnp.exp(v) - 1.0))
  xc = jnp.concatenate(cols, axis=1)
  h2 = jnp.dot(xc, wo_ref[...], preferred_element_type=jnp.float32)
  ao = ao_ref[...]
  sl2 = jnp.sum(h2 * ao[:, :NCLASS], axis=1, keepdims=True)
  sr2 = jnp.sum(h2 * ao[:, NCLASS:], axis=1, keepdims=True)
  one = jnp.ones((h2.shape[0], 1), jnp.float32)
  hp2_ref[...] = jnp.concatenate(
      [h2, one, sr2, sl2, jnp.zeros((h2.shape[0], 109), jnp.float32)], axis=1)


def _tc2(acc, wo, ao):
  return pl.pallas_call(
      _tc2_body,
      grid=(_GRID,),
      in_specs=[
          pl.BlockSpec((2, _BLK, 128), lambda i: (0, i, 0)),
          pl.BlockSpec((NHID * NHEADS, NCLASS), lambda i: (0, 0)),
          pl.BlockSpec((1, 2 * NCLASS), lambda i: (0, 0)),
      ],
      out_specs=pl.BlockSpec((_BLK, 128), lambda i: (i, 0)),
      out_shape=jax.ShapeDtypeStruct((N, 128), jnp.float32),
  )(acc, wo, ao)


def _tc3_body(acc_ref, out_ref):
  A = acc_ref[0] + acc_ref[1]
  out_ref[...] = A[:, :NCLASS] / (A[:, NCLASS:NCLASS + 1] + 1e-16)


def _tc3(acc):
  return pl.pallas_call(
      _tc3_body,
      grid=(_GRID,),
      in_specs=[pl.BlockSpec((2, _BLK, 128), lambda i: (0, i, 0))],
      out_specs=pl.BlockSpec((_BLK, NCLASS), lambda i: (i, 0)),
      out_shape=jax.ShapeDtypeStruct((N, NCLASS), jnp.float32),
  )(acc)


def _make_sc_edge(heads):
  def body(row_h, col_h, hp_h, z_h, out_h, rv, cv, hv, hvr, cb, acc, sem1,
           sem2):
    c = lax.axis_index("c")
    s = lax.axis_index("s")
    w = c * NS + s
    lanes = lax.iota(jnp.int32, 16)

    pltpu.sync_copy(z_h, cb)
    r0 = s * RPS
    for off in range(0, RPS, B):
      sz = min(B, RPS - off)
      pltpu.sync_copy(cb.at[pl.ds(0, sz)], acc.at[pl.ds(r0 + off, sz)])

    @pl.when(s == NS - 1)
    def _():
      pltpu.sync_copy(cb.at[pl.ds(0, REM)], acc.at[pl.ds(NS * RPS, REM)])

    plsc.subcore_barrier()

    def batch_body(t, carry):
      b = w + t * (NC * NS)

      @pl.when(b < NB)
      def _():
        pltpu.sync_copy(row_h.at[pl.ds(b, 1)], rv)
        pltpu.sync_copy(col_h.at[pl.ds(b, 1)], cv)
        d1 = pltpu.async_copy(hp_h.at[cv.at[0]], hv, sem1)
        d2 = pltpu.async_copy(hp_h.at[rv.at[0]], hvr, sem2)
        d1.wait()
        d2.wait()
        for g in range(B // 16):
          e = g * 16 + lanes
          for (slc, src, pairs, dcol) in heads:
            s1 = plsc.load_gather(hvr, [e, jnp.full((16,), slc, jnp.int32)])
            s2 = plsc.load_gather(hv, [e, jnp.full((16,), src, jnp.int32)])
            v = s1 + s2
            lr = jnp.where(v > 0, v, ALPHA * v)
            wgt = jnp.exp(-lr)
            for (hcol, ocol) in pairs:
              hc = plsc.load_gather(hv, [e, jnp.full((16,), hcol, jnp.int32)])
              plsc.store_scatter(cb, [e, jnp.full((16,), ocol, jnp.int32)],
                                 wgt * hc)
            plsc.store_scatter(cb, [e, jnp.full((16,), dcol, jnp.int32)], wgt)
        pltpu.async_copy(cb, acc.at[rv.at[0]], sem1, add=True).wait()

      return carry

    lax.fori_loop(0, (NB + NC * NS - 1) // (NC * NS), batch_body, 0)
    plsc.subcore_barrier()
    pltpu.sync_copy(acc.at[pl.ds(r0, RPS)], out_h.at[c, pl.ds(r0, RPS)])

    @pl.when(s == NS - 1)
    def _():
      pltpu.sync_copy(acc.at[pl.ds(NS * RPS, REM)],
                      out_h.at[c, pl.ds(NS * RPS, REM)])

  mesh = plsc.VectorSubcoreMesh(
      core_axis_name="c", subcore_axis_name="s", num_cores=NC, num_subcores=NS)
  return pl.kernel(
      body,
      out_type=jax.ShapeDtypeStruct((NC, N, 128), jnp.float32),
      mesh=mesh,
      compiler_params=pltpu.CompilerParams(needs_layout_passes=False),
      scratch_types=[
          pltpu.VMEM((1, B), jnp.int32),
          pltpu.VMEM((1, B), jnp.int32),
          pltpu.VMEM((B, 128), jnp.float32),
          pltpu.VMEM((B, 128), jnp.float32),
          pltpu.VMEM((B, 128), jnp.float32),
          pltpu.VMEM_SHARED((N, 128), jnp.float32),
          pltpu.SemaphoreType.DMA,
          pltpu.SemaphoreType.DMA,
      ],
  )


_HEADS1 = [
    (88 + i, 80 + i, [(10 * i + k, 10 * i + k) for k in range(NHID)],
     10 * i + 8)
    for i in range(NHEADS)
]
_HEADS2 = [(NCLASS + 2, NCLASS + 1, [(k, k) for k in range(NCLASS)], NCLASS)]

_sc_edge1 = _make_sc_edge(_HEADS1)
_sc_edge2 = _make_sc_edge(_HEADS2)


@jax.jit
def kernel(x, edge_index, W, a, W_out, a_out):
  row = edge_index[0]
  col = edge_index[1]

  wt = jnp.transpose(W, (1, 0, 2))
  wp = jnp.pad(wt, ((0, 0), (0, 0), (0, 2))).reshape(NFEAT, 80)
  I = jnp.arange(NHEADS)
  K = jnp.arange(NHID)
  al = jnp.zeros((NHEADS, 10, NHEADS), jnp.float32)
  al = al.at[I[:, None], K[None, :], I[:, None]].set(a[:, :NHID])
  al = al.reshape(80, NHEADS)
  ar = jnp.zeros((NHEADS, 10, NHEADS), jnp.float32)
  ar = ar.at[I[:, None], K[None, :], I[:, None]].set(a[:, NHID:])
  ar = ar.reshape(80, NHEADS)

  row2 = row.reshape(NB, B)
  col2 = col.reshape(NB, B)
  z = jnp.zeros((B, 128), jnp.float32)
  hp = _tc1(x, wp, al, ar)
  acc1 = _sc_edge1(row2, col2, hp, z)
  hp2 = _tc2(acc1, W_out, a_out.reshape(1, 2 * NCLASS))
  acc2 = _sc_edge2(row2, col2, hp2, z)
  return _tc3(acc2)
